# baseline (speedup 1.0000x reference)
import functools

import jax
import jax.numpy as jnp
from jax import lax
from jax.experimental import pallas as pl
from jax.experimental.pallas import tpu as pltpu

_LN_EPS = 1e-5
_NEG_INF = -1e30
_VMEM_LIMIT = 56 * 1024 * 1024


def _ln(x, g, b):
    mu = jnp.mean(x, axis=-1, keepdims=True)
    var = jnp.mean(jnp.square(x - mu), axis=-1, keepdims=True)
    return (x - mu) * lax.rsqrt(var + _LN_EPS) * g + b


def _gelu(x):
    return 0.5 * x * (1.0 + lax.erf(x * 0.7071067811865476))


def _dot(a, b):
    return jnp.dot(a.astype(jnp.bfloat16), b.astype(jnp.bfloat16),
                   preferred_element_type=jnp.float32)


def _vit_kernel(patches_ref, cls_ref, pos_ref,
                pg1_ref, pb1_ref, pw_ref, pbw_ref, pg2_ref, pb2_ref,
                ga_ref, ba_ref, wqkv_ref, wo_ref, bo_ref,
                gf_ref, bf_ref, w1_ref, b1_ref, w2_ref, b2_ref,
                hg_ref, hb_ref, hw_ref, hbw_ref,
                o_ref, x_vmem, attn_vmem,
                *, heads, dim_head, scale, n_real):
    layer = pl.program_id(1)
    n_layers = pl.num_programs(1)
    Bt, N, D = x_vmem.shape
    inner = heads * dim_head

    @pl.when(layer == 0)
    def _embed():
        p = patches_ref[...].reshape(Bt * N, patches_ref.shape[-1])
        pn = _ln(p, pg1_ref[...], pb1_ref[...])
        y = _dot(pn, pw_ref[...]) + pbw_ref[...]
        pe = _ln(y, pg2_ref[...], pb2_ref[...]).reshape(Bt, N, D)
        row = lax.broadcasted_iota(jnp.int32, (1, N, 1), 1)
        cls = cls_ref[...].reshape(1, 1, D)
        x_vmem[...] = jnp.where(row == 0, cls, pe) + pos_ref[...].reshape(1, N, D)

    x = x_vmem[...].reshape(Bt * N, D)

    xa = _ln(x, ga_ref[...], ba_ref[...]).astype(jnp.bfloat16)
    qkv = jnp.dot(xa, wqkv_ref[...],
                  preferred_element_type=jnp.float32).astype(jnp.bfloat16)

    col = lax.broadcasted_iota(jnp.int32, (1, 1, N), 2)
    kmask = jnp.where(col < n_real, 0.0, _NEG_INF).astype(jnp.float32)

    for h in range(heads):
        q = qkv[:, h * dim_head:(h + 1) * dim_head].reshape(Bt, N, dim_head)
        k = qkv[:, inner + h * dim_head:
                   inner + (h + 1) * dim_head].reshape(Bt, N, dim_head)
        v = qkv[:, 2 * inner + h * dim_head:
                   2 * inner + (h + 1) * dim_head].reshape(Bt, N, dim_head)
        s = jnp.einsum('bnd,bmd->bnm', q, k,
                       preferred_element_type=jnp.float32) * scale + kmask
        s = s - jnp.max(s, axis=-1, keepdims=True)
        e = jnp.exp(s)
        denom = pl.reciprocal(jnp.sum(e, axis=-1, keepdims=True), approx=True)
        av = jnp.einsum('bnm,bmd->bnd', e.astype(jnp.bfloat16), v,
                        preferred_element_type=jnp.float32)
        attn_vmem[..., h * dim_head:(h + 1) * dim_head] = (
            av * denom).astype(jnp.bfloat16)

    attn = attn_vmem[...].reshape(Bt * N, inner)
    x = x + jnp.dot(attn, wo_ref[...],
                    preferred_element_type=jnp.float32) + bo_ref[...]

    xf = _ln(x, gf_ref[...], bf_ref[...]).astype(jnp.bfloat16)
    h1 = _gelu(jnp.dot(xf, w1_ref[...],
                       preferred_element_type=jnp.float32) + b1_ref[...]
               ).astype(jnp.bfloat16)
    x = x + jnp.dot(h1, w2_ref[...],
                    preferred_element_type=jnp.float32) + b2_ref[...]

    x_vmem[...] = x.reshape(Bt, N, D)

    @pl.when(layer == n_layers - 1)
    def _head():
        xc = x_vmem[:, 0, :]
        xh = _ln(xc, hg_ref[...], hb_ref[...]).astype(jnp.bfloat16)
        o_ref[...] = (jnp.dot(xh, hw_ref[...],
                              preferred_element_type=jnp.float32)
                      + hbw_ref[...])[None]


@jax.jit
def kernel(img, patch_g1, patch_b1, patch_w, patch_bw, patch_g2, patch_b2,
           cls_token, pos_embedding,
           tb_ga, tb_ba, tb_wqkv, tb_wo, tb_bo, tb_gf, tb_bf,
           tb_w1, tb_b1, tb_w2, tb_b2,
           head_g, head_b, head_w, head_bw):
    B, C, H, W = img.shape
    ps = 16
    hh, ww = H // ps, W // ps
    Np = hh * ww
    Pd = ps * ps * C
    D = patch_w.shape[1]
    depth = tb_wqkv.shape[0]
    inner = tb_wqkv.shape[2] // 3
    heads = 12
    dim_head = inner // heads
    mlp = tb_w1.shape[2]

    patches = img.reshape(B, C, hh, ps, ww, ps).transpose(0, 2, 4, 1, 3, 5)
    patches = patches.reshape(B, Np, Pd)
    perm = (jnp.arange(C)[:, None] + C * jnp.arange(ps * ps)[None, :]
            ).reshape(Pd)
    patch_w = patch_w[perm]
    patch_g1 = patch_g1[:, perm]
    patch_b1 = patch_b1[:, perm]
    n_real = Np + 1
    N = ((n_real + 7) // 8) * 8
    patches = jnp.pad(patches, ((0, 0), (1, N - n_real), (0, 0)))
    pos = jnp.pad(pos_embedding, ((0, 0), (0, N - n_real), (0, 0)))

    nc = head_w.shape[1]
    ncp = ((nc + 127) // 128) * 128
    hw = jnp.pad(head_w, ((0, 0), (0, ncp - nc)))
    hbw = jnp.pad(head_bw, ((0, 0), (0, ncp - nc)))

    Bt = min(4, B)
    assert B % Bt == 0

    def cspec(*shape):
        nd = len(shape)
        return pl.BlockSpec(shape, lambda b, l, _nd=nd: (0,) * _nd)

    def wspec(*shape):
        nd = len(shape)
        return pl.BlockSpec((None,) + shape,
                            lambda b, l, _nd=nd: (l,) + (0,) * _nd)

    kern = functools.partial(_vit_kernel, heads=heads, dim_head=dim_head,
                             scale=dim_head ** -0.5, n_real=n_real)
    out = pl.pallas_call(
        kern,
        out_shape=jax.ShapeDtypeStruct((B // Bt, Bt, ncp), jnp.float32),
        grid=(B // Bt, depth),
        in_specs=[pl.BlockSpec((Bt, N, Pd), lambda b, l: (b, 0, 0)),
                  cspec(1, D),
                  pl.BlockSpec((None, N, D), lambda b, l: (0, 0, 0)),
                  cspec(1, Pd), cspec(1, Pd),
                  cspec(Pd, D), cspec(1, D),
                  cspec(1, D), cspec(1, D),
                  wspec(1, D), wspec(1, D),
                  wspec(D, 3 * inner), wspec(inner, D), wspec(1, D),
                  wspec(1, D), wspec(1, D),
                  wspec(D, mlp), wspec(1, mlp),
                  wspec(mlp, D), wspec(1, D),
                  cspec(1, D), cspec(1, D),
                  cspec(D, ncp), cspec(1, ncp)],
        out_specs=pl.BlockSpec((1, Bt, ncp), lambda b, l: (b, 0, 0)),
        scratch_shapes=[pltpu.VMEM((Bt, N, D), jnp.float32),
                        pltpu.VMEM((Bt, N, inner), jnp.bfloat16)],
        compiler_params=pltpu.CompilerParams(
            dimension_semantics=("arbitrary", "arbitrary"),
            vmem_limit_bytes=_VMEM_LIMIT),
    )(patches, cls_token.reshape(1, D), pos,
      patch_g1, patch_b1, patch_w, patch_bw, patch_g2, patch_b2,
      tb_ga, tb_ba, tb_wqkv, tb_wo, tb_bo, tb_gf, tb_bf,
      tb_w1, tb_b1, tb_w2, tb_b2,
      head_g, head_b, hw, hbw)
    return out.reshape(B, ncp)[:, :nc]


# speedup vs baseline: 1.0216x; 1.0216x over previous
import functools

import jax
import jax.numpy as jnp
from jax import lax
from jax.experimental import pallas as pl
from jax.experimental.pallas import tpu as pltpu

_LN_EPS = 1e-5
_NEG_INF = -1e30
_VMEM_LIMIT = 56 * 1024 * 1024


def _ln(x, g, b):
    mu = jnp.mean(x, axis=-1, keepdims=True)
    var = jnp.mean(jnp.square(x - mu), axis=-1, keepdims=True)
    return (x - mu) * lax.rsqrt(var + _LN_EPS) * g + b


def _gelu(x):
    return 0.5 * x * (1.0 + lax.erf(x * 0.7071067811865476))


def _dot(a, b):
    return jnp.dot(a.astype(jnp.bfloat16), b.astype(jnp.bfloat16),
                   preferred_element_type=jnp.float32)


def _vit_kernel(patches_ref, cls_ref, pos_ref,
                pg1_ref, pb1_ref, pw_ref, pbw_ref, pg2_ref, pb2_ref,
                ga_ref, ba_ref, wqkv_ref, wo_ref, bo_ref,
                gf_ref, bf_ref, w1_ref, b1_ref, w2_ref, b2_ref,
                hg_ref, hb_ref, hw_ref, hbw_ref,
                o_ref, x_vmem, attn_vmem,
                *, heads, dim_head, scale, n_real):
    layer = pl.program_id(1)
    n_layers = pl.num_programs(1)
    Bt, N, D = x_vmem.shape
    inner = heads * dim_head

    @pl.when(layer == 0)
    def _embed():
        p = patches_ref[...].reshape(Bt * N, patches_ref.shape[-1])
        pn = _ln(p, pg1_ref[...], pb1_ref[...])
        y = _dot(pn, pw_ref[...]) + pbw_ref[...]
        pe = _ln(y, pg2_ref[...], pb2_ref[...]).reshape(Bt, N, D)
        row = lax.broadcasted_iota(jnp.int32, (1, N, 1), 1)
        cls = cls_ref[...].reshape(1, 1, D)
        x_vmem[...] = jnp.where(row == 0, cls, pe) + pos_ref[...].reshape(1, N, D)

    x = x_vmem[...].reshape(Bt * N, D)

    xa = _ln(x, ga_ref[...], ba_ref[...]).astype(jnp.bfloat16)
    qkv = jnp.dot(xa, wqkv_ref[...],
                  preferred_element_type=jnp.float32).astype(jnp.bfloat16)

    col = lax.broadcasted_iota(jnp.int32, (1, 1, N), 2)
    kmask = jnp.where(col < n_real, 0.0, _NEG_INF).astype(jnp.float32)

    for h in range(heads):
        q = qkv[:, h * dim_head:(h + 1) * dim_head].reshape(Bt, N, dim_head)
        k = qkv[:, inner + h * dim_head:
                   inner + (h + 1) * dim_head].reshape(Bt, N, dim_head)
        v = qkv[:, 2 * inner + h * dim_head:
                   2 * inner + (h + 1) * dim_head].reshape(Bt, N, dim_head)
        s = jnp.einsum('bnd,bmd->bnm', q, k,
                       preferred_element_type=jnp.float32) * scale + kmask
        s = s - jnp.max(s, axis=-1, keepdims=True)
        e = jnp.exp(s)
        denom = pl.reciprocal(jnp.sum(e, axis=-1, keepdims=True), approx=True)
        av = jnp.einsum('bnm,bmd->bnd', e.astype(jnp.bfloat16), v,
                        preferred_element_type=jnp.float32)
        attn_vmem[..., h * dim_head:(h + 1) * dim_head] = (
            av * denom).astype(jnp.bfloat16)

    attn = attn_vmem[...].reshape(Bt * N, inner)
    x = x + jnp.dot(attn, wo_ref[...],
                    preferred_element_type=jnp.float32) + bo_ref[...]

    xf = _ln(x, gf_ref[...], bf_ref[...]).astype(jnp.bfloat16)
    h1 = _gelu(jnp.dot(xf, w1_ref[...],
                       preferred_element_type=jnp.float32) + b1_ref[...]
               ).astype(jnp.bfloat16)
    x = x + jnp.dot(h1, w2_ref[...],
                    preferred_element_type=jnp.float32) + b2_ref[...]

    x_vmem[...] = x.reshape(Bt, N, D)

    @pl.when(layer == n_layers - 1)
    def _head():
        xc = x_vmem[:, 0, :]
        xh = _ln(xc, hg_ref[...], hb_ref[...]).astype(jnp.bfloat16)
        o_ref[...] = (jnp.dot(xh, hw_ref[...],
                              preferred_element_type=jnp.float32)
                      + hbw_ref[...])[None]


@jax.jit
def kernel(img, patch_g1, patch_b1, patch_w, patch_bw, patch_g2, patch_b2,
           cls_token, pos_embedding,
           tb_ga, tb_ba, tb_wqkv, tb_wo, tb_bo, tb_gf, tb_bf,
           tb_w1, tb_b1, tb_w2, tb_b2,
           head_g, head_b, head_w, head_bw):
    B, C, H, W = img.shape
    ps = 16
    hh, ww = H // ps, W // ps
    Np = hh * ww
    Pd = ps * ps * C
    D = patch_w.shape[1]
    depth = tb_wqkv.shape[0]
    inner = tb_wqkv.shape[2] // 3
    heads = 12
    dim_head = inner // heads
    mlp = tb_w1.shape[2]

    patches = img.reshape(B, C, hh, ps, ww, ps).transpose(0, 2, 4, 1, 3, 5)
    patches = patches.reshape(B, Np, Pd)
    patch_w = patch_w.reshape(ps * ps, C, D).transpose(1, 0, 2).reshape(Pd, D)
    patch_g1 = patch_g1.reshape(ps * ps, C).T.reshape(1, Pd)
    patch_b1 = patch_b1.reshape(ps * ps, C).T.reshape(1, Pd)
    n_real = Np + 1
    N = ((n_real + 7) // 8) * 8
    patches = jnp.pad(patches, ((0, 0), (1, N - n_real), (0, 0)))
    pos = jnp.pad(pos_embedding, ((0, 0), (0, N - n_real), (0, 0)))

    nc = head_w.shape[1]
    ncp = ((nc + 127) // 128) * 128
    hw = jnp.pad(head_w, ((0, 0), (0, ncp - nc)))
    hbw = jnp.pad(head_bw, ((0, 0), (0, ncp - nc)))

    Bt = min(4, B)
    assert B % Bt == 0

    def cspec(*shape):
        nd = len(shape)
        return pl.BlockSpec(shape, lambda b, l, _nd=nd: (0,) * _nd)

    def wspec(*shape):
        nd = len(shape)
        return pl.BlockSpec((None,) + shape,
                            lambda b, l, _nd=nd: (l,) + (0,) * _nd)

    kern = functools.partial(_vit_kernel, heads=heads, dim_head=dim_head,
                             scale=dim_head ** -0.5, n_real=n_real)
    out = pl.pallas_call(
        kern,
        out_shape=jax.ShapeDtypeStruct((B // Bt, Bt, ncp), jnp.float32),
        grid=(B // Bt, depth),
        in_specs=[pl.BlockSpec((Bt, N, Pd), lambda b, l: (b, 0, 0)),
                  cspec(1, D),
                  pl.BlockSpec((None, N, D), lambda b, l: (0, 0, 0)),
                  cspec(1, Pd), cspec(1, Pd),
                  cspec(Pd, D), cspec(1, D),
                  cspec(1, D), cspec(1, D),
                  wspec(1, D), wspec(1, D),
                  wspec(D, 3 * inner), wspec(inner, D), wspec(1, D),
                  wspec(1, D), wspec(1, D),
                  wspec(D, mlp), wspec(1, mlp),
                  wspec(mlp, D), wspec(1, D),
                  cspec(1, D), cspec(1, D),
                  cspec(D, ncp), cspec(1, ncp)],
        out_specs=pl.BlockSpec((1, Bt, ncp), lambda b, l: (b, 0, 0)),
        scratch_shapes=[pltpu.VMEM((Bt, N, D), jnp.float32),
                        pltpu.VMEM((Bt, N, inner), jnp.bfloat16)],
        compiler_params=pltpu.CompilerParams(
            dimension_semantics=("arbitrary", "arbitrary"),
            vmem_limit_bytes=_VMEM_LIMIT),
    )(patches, cls_token.reshape(1, D), pos,
      patch_g1, patch_b1, patch_w, patch_bw, patch_g2, patch_b2,
      tb_ga, tb_ba, tb_wqkv, tb_wo, tb_bo, tb_gf, tb_bf,
      tb_w1, tb_b1, tb_w2, tb_b2,
      head_g, head_b, hw, hbw)
    return out.reshape(B, ncp)[:, :nc]


# speedup vs baseline: 1.1243x; 1.1006x over previous
import functools

import jax
import jax.numpy as jnp
from jax import lax
from jax.experimental import pallas as pl
from jax.experimental.pallas import tpu as pltpu

_LN_EPS = 1e-5
_NEG_INF = -1e30
_VMEM_LIMIT = 56 * 1024 * 1024


def _ln(x, g, b):
    mu = jnp.mean(x, axis=-1, keepdims=True)
    var = jnp.mean(jnp.square(x - mu), axis=-1, keepdims=True)
    return (x - mu) * lax.rsqrt(var + _LN_EPS) * g + b


def _gelu(x):
    return 0.5 * x * (1.0 + lax.erf(x * 0.7071067811865476))


def _dot(a, b):
    return jnp.dot(a.astype(jnp.bfloat16), b.astype(jnp.bfloat16),
                   preferred_element_type=jnp.float32)


def _vit_kernel(patches_ref, cls_ref, pos_ref,
                pg1_ref, pb1_ref, pw_ref, pbw_ref, pg2_ref, pb2_ref,
                ga_ref, ba_ref, wqkv_ref, wo_ref, bo_ref,
                gf_ref, bf_ref, w1_ref, b1_ref, w2_ref, b2_ref,
                hg_ref, hb_ref, hw_ref, hbw_ref,
                o_ref, x_vmem, attn_vmem,
                *, heads, dim_head, scale, n_real):
    layer = pl.program_id(1)
    n_layers = pl.num_programs(1)
    Bt, N, D = x_vmem.shape
    inner = heads * dim_head

    @pl.when(layer == 0)
    def _embed():
        p = patches_ref[...].reshape(Bt * N, patches_ref.shape[-1])
        pn = _ln(p, pg1_ref[...], pb1_ref[...])
        y = _dot(pn, pw_ref[...]) + pbw_ref[...]
        pe = _ln(y, pg2_ref[...], pb2_ref[...]).reshape(Bt, N, D)
        row = lax.broadcasted_iota(jnp.int32, (1, N, 1), 1)
        cls = cls_ref[...].reshape(1, 1, D)
        x_vmem[...] = jnp.where(row == 0, cls, pe) + pos_ref[...].reshape(1, N, D)

    x = x_vmem[...].reshape(Bt * N, D)

    xa = _ln(x, ga_ref[...], ba_ref[...]).astype(jnp.bfloat16)
    qkv = jnp.dot(xa, wqkv_ref[...],
                  preferred_element_type=jnp.float32).astype(jnp.bfloat16)

    col = lax.broadcasted_iota(jnp.int32, (1, 1, N), 2)
    kmask = jnp.where(col < n_real, 0.0, _NEG_INF).astype(jnp.float32)

    for h in range(heads):
        q = qkv[:, h * dim_head:(h + 1) * dim_head].reshape(Bt, N, dim_head)
        k = qkv[:, inner + h * dim_head:
                   inner + (h + 1) * dim_head].reshape(Bt, N, dim_head)
        v = qkv[:, 2 * inner + h * dim_head:
                   2 * inner + (h + 1) * dim_head].reshape(Bt, N, dim_head)
        s = jnp.einsum('bnd,bmd->bnm', q, k,
                       preferred_element_type=jnp.float32) * scale + kmask
        e = jnp.exp(s)
        p_ = (e * pl.reciprocal(jnp.sum(e, axis=-1, keepdims=True),
                                approx=True)).astype(jnp.bfloat16)
        attn_vmem[..., h * dim_head:(h + 1) * dim_head] = jnp.einsum(
            'bnm,bmd->bnd', p_, v,
            preferred_element_type=jnp.float32).astype(jnp.bfloat16)

    attn = attn_vmem[...].reshape(Bt * N, inner)
    x = x + jnp.dot(attn, wo_ref[...],
                    preferred_element_type=jnp.float32) + bo_ref[...]

    xf = _ln(x, gf_ref[...], bf_ref[...]).astype(jnp.bfloat16)
    h1 = _gelu(jnp.dot(xf, w1_ref[...],
                       preferred_element_type=jnp.float32) + b1_ref[...]
               ).astype(jnp.bfloat16)
    x = x + jnp.dot(h1, w2_ref[...],
                    preferred_element_type=jnp.float32) + b2_ref[...]

    x_vmem[...] = x.reshape(Bt, N, D)

    @pl.when(layer == n_layers - 1)
    def _head():
        xc = x_vmem[:, 0, :]
        xh = _ln(xc, hg_ref[...], hb_ref[...]).astype(jnp.bfloat16)
        o_ref[...] = (jnp.dot(xh, hw_ref[...],
                              preferred_element_type=jnp.float32)
                      + hbw_ref[...])[None]


@jax.jit
def kernel(img, patch_g1, patch_b1, patch_w, patch_bw, patch_g2, patch_b2,
           cls_token, pos_embedding,
           tb_ga, tb_ba, tb_wqkv, tb_wo, tb_bo, tb_gf, tb_bf,
           tb_w1, tb_b1, tb_w2, tb_b2,
           head_g, head_b, head_w, head_bw):
    B, C, H, W = img.shape
    ps = 16
    hh, ww = H // ps, W // ps
    Np = hh * ww
    Pd = ps * ps * C
    D = patch_w.shape[1]
    depth = tb_wqkv.shape[0]
    inner = tb_wqkv.shape[2] // 3
    heads = 12
    dim_head = inner // heads
    mlp = tb_w1.shape[2]

    patches = img.reshape(B, C, hh, ps, ww, ps).transpose(0, 2, 4, 3, 5, 1)
    patches = patches.reshape(B, Np, Pd)
    n_real = Np + 1
    N = ((n_real + 7) // 8) * 8
    patches = jnp.pad(patches, ((0, 0), (1, N - n_real), (0, 0)))
    pos = jnp.pad(pos_embedding, ((0, 0), (0, N - n_real), (0, 0)))

    nc = head_w.shape[1]
    ncp = ((nc + 127) // 128) * 128
    hw = jnp.pad(head_w, ((0, 0), (0, ncp - nc)))
    hbw = jnp.pad(head_bw, ((0, 0), (0, ncp - nc)))

    Bt = min(4, B)
    assert B % Bt == 0

    def cspec(*shape):
        nd = len(shape)
        return pl.BlockSpec(shape, lambda b, l, _nd=nd: (0,) * _nd)

    def wspec(*shape):
        nd = len(shape)
        return pl.BlockSpec((None,) + shape,
                            lambda b, l, _nd=nd: (l,) + (0,) * _nd)

    kern = functools.partial(_vit_kernel, heads=heads, dim_head=dim_head,
                             scale=dim_head ** -0.5, n_real=n_real)
    out = pl.pallas_call(
        kern,
        out_shape=jax.ShapeDtypeStruct((B // Bt, Bt, ncp), jnp.float32),
        grid=(B // Bt, depth),
        in_specs=[pl.BlockSpec((Bt, N, Pd), lambda b, l: (b, 0, 0)),
                  cspec(1, D),
                  pl.BlockSpec((None, N, D), lambda b, l: (0, 0, 0)),
                  cspec(1, Pd), cspec(1, Pd),
                  cspec(Pd, D), cspec(1, D),
                  cspec(1, D), cspec(1, D),
                  wspec(1, D), wspec(1, D),
                  wspec(D, 3 * inner), wspec(inner, D), wspec(1, D),
                  wspec(1, D), wspec(1, D),
                  wspec(D, mlp), wspec(1, mlp),
                  wspec(mlp, D), wspec(1, D),
                  cspec(1, D), cspec(1, D),
                  cspec(D, ncp), cspec(1, ncp)],
        out_specs=pl.BlockSpec((1, Bt, ncp), lambda b, l: (b, 0, 0)),
        scratch_shapes=[pltpu.VMEM((Bt, N, D), jnp.float32),
                        pltpu.VMEM((Bt, N, inner), jnp.bfloat16)],
        compiler_params=pltpu.CompilerParams(
            dimension_semantics=("arbitrary", "arbitrary"),
            vmem_limit_bytes=_VMEM_LIMIT),
    )(patches, cls_token.reshape(1, D), pos,
      patch_g1, patch_b1, patch_w, patch_bw, patch_g2, patch_b2,
      tb_ga, tb_ba, tb_wqkv, tb_wo, tb_bo, tb_gf, tb_bf,
      tb_w1, tb_b1, tb_w2, tb_b2,
      head_g, head_b, hw, hbw)
    return out.reshape(B, ncp)[:, :nc]
